# baseline (speedup 1.0000x reference)
import functools

import jax
import jax.numpy as jnp
from jax import lax
from jax.experimental import pallas as pl
from jax.experimental.pallas import tpu as pltpu
from jax.experimental.pallas import tpu_sc as plsc

NC = 2
NS = 16
NW = NC * NS
NB = 2


def _mm1_body(x_ref, w_ref, b_ref, o_ref):
    acc = jnp.dot(x_ref[...], w_ref[...], preferred_element_type=jnp.float32)
    o_ref[...] = jnp.maximum(acc + b_ref[...], 0.0)


def _mm2_body(p_ref, h_ref, w_ref, b_ref, o_ref):
    agg = p_ref[0] + p_ref[1]
    acc = jnp.dot(agg, w_ref[...], preferred_element_type=jnp.float32)
    o_ref[...] = jnp.maximum(acc + b_ref[...], 0.0) + h_ref[...]


def _sc_agg(h, src3, dst3, pad_n, k, ch0, ch1):
    n, d = h.shape
    chmax = max(ch0, ch1)
    rpt = pad_n // NS
    mesh = plsc.VectorSubcoreMesh(core_axis_name="c", subcore_axis_name="s")

    @functools.partial(
        pl.kernel,
        out_type=jax.ShapeDtypeStruct((NC, pad_n, d), jnp.float32),
        mesh=mesh,
        scratch_types=[
            pltpu.VMEM((chmax + NB, k), jnp.int32),
            pltpu.VMEM((k, d), jnp.float32),
            pltpu.VMEM((k, d), jnp.float32),
            pltpu.VMEM_SHARED((pad_n, d), jnp.float32),
            pltpu.SemaphoreType.DMA,
            pltpu.SemaphoreType.DMA,
            pltpu.SemaphoreType.DMA,
            pltpu.SemaphoreType.DMA,
        ],
    )
    def agg_kernel(h_hbm, src_hbm, dst_hbm, out_hbm,
                   src_v, rows0, rows1, shared,
                   gsem0, gsem1, dsem0, dsem1):
        cid = lax.axis_index("c")
        sid = lax.axis_index("s")
        row0 = sid * rpt
        chc = jnp.where(cid == 0, ch0, ch1)

        @pl.loop(0, k)
        def _(r):
            @pl.loop(0, d, step=16)
            def _(c0):
                rows0[r, pl.ds(c0, 16)] = jnp.zeros((16,), jnp.float32)

        nfull, rem = divmod(rpt, k)

        @pl.loop(0, nfull * k, step=k)
        def _(r):
            pltpu.sync_copy(rows0, shared.at[pl.ds(row0 + r, k)])

        if rem:
            pltpu.sync_copy(rows0.at[pl.ds(0, rem)],
                            shared.at[pl.ds(row0 + nfull * k, rem)])

        gwid = cid * NS + sid
        pltpu.sync_copy(src_hbm.at[gwid], src_v.at[pl.ds(0, chmax)])

        plsc.subcore_barrier()

        rows = (rows0, rows1)
        gsem = (gsem0, gsem1)
        dsem = (dsem0, dsem1)
        for b in range(NB):
            pltpu.async_copy(dst_hbm.at[gwid, b], src_v.at[chmax + b], dsem[b])
            pltpu.async_copy(h_hbm.at[src_v.at[b]], rows[b], gsem[b])

        @pl.loop(0, chc, step=NB)
        def _(j):
            for b in range(NB):
                pltpu.make_async_copy(dst_hbm.at[gwid, j + b],
                                      src_v.at[chmax + b], dsem[b]).wait()
                pltpu.make_async_copy(h_hbm.at[src_v.at[j + b]],
                                      rows[b], gsem[b]).wait()
                pltpu.sync_copy(rows[b], shared.at[src_v.at[chmax + b]],
                                add=True)

                @pl.when(j + b + NB < chc)
                def _():
                    pltpu.async_copy(dst_hbm.at[gwid, j + b + NB],
                                     src_v.at[chmax + b], dsem[b])
                    pltpu.async_copy(h_hbm.at[src_v.at[j + b + NB]],
                                     rows[b], gsem[b])

        plsc.subcore_barrier()

        pltpu.sync_copy(shared.at[pl.ds(row0, rpt)],
                        out_hbm.at[cid, pl.ds(row0, rpt)])

    return agg_kernel(h, src3, dst3)


def _split_edges(idx, e, k, ch0, ch1, pad_val):
    chmax = max(ch0, ch1)
    e0 = NS * ch0 * k
    cap1 = NS * ch1 * k
    p0 = idx[:e0].reshape(NS, ch0, k)
    p1 = jnp.concatenate(
        [idx[e0:], jnp.full((e0 + cap1 - e,), pad_val, jnp.int32)]
    ).reshape(NS, ch1, k)
    p0 = jnp.pad(p0, ((0, 0), (0, chmax - ch0), (0, 0)),
                 constant_values=pad_val)
    p1 = jnp.pad(p1, ((0, 0), (0, chmax - ch1), (0, 0)),
                 constant_values=pad_val)
    return jnp.concatenate([p0, p1], axis=0)


def kernel(x, edge_index, W1, b1, W2, b2):
    n, d = x.shape
    e = edge_index.shape[1]

    k = 88
    ch0, ch1 = 90, 138
    assert NS * (ch0 + ch1) * k >= e and ch0 % NB == 0 and ch1 % NB == 0
    pad_n = ((n + 127) // 128) * 128

    src = _split_edges(edge_index[0], e, k, ch0, ch1, 0)
    dst = _split_edges(edge_index[1], e, k, ch0, ch1, n)

    bn = 1000
    grid = (n // bn,)
    b1r = b1.reshape(1, d)
    b2r = b2.reshape(1, d)

    h = pl.pallas_call(
        _mm1_body,
        grid=grid,
        in_specs=[
            pl.BlockSpec((bn, d), lambda i: (i, 0)),
            pl.BlockSpec((d, d), lambda i: (0, 0)),
            pl.BlockSpec((1, d), lambda i: (0, 0)),
        ],
        out_specs=pl.BlockSpec((bn, d), lambda i: (i, 0)),
        out_shape=jax.ShapeDtypeStruct((n, d), jnp.float32),
    )(x, W1, b1r)

    partials = _sc_agg(h, src, dst, pad_n, k, ch0, ch1)

    out = pl.pallas_call(
        _mm2_body,
        grid=grid,
        in_specs=[
            pl.BlockSpec((NC, bn, d), lambda i: (0, i, 0)),
            pl.BlockSpec((bn, d), lambda i: (i, 0)),
            pl.BlockSpec((d, d), lambda i: (0, 0)),
            pl.BlockSpec((1, d), lambda i: (0, 0)),
        ],
        out_specs=pl.BlockSpec((bn, d), lambda i: (i, 0)),
        out_shape=jax.ShapeDtypeStruct((n, d), jnp.float32),
    )(partials, h, W2, b2r)

    return out


# speedup vs baseline: 1.1070x; 1.1070x over previous
import functools

import jax
import jax.numpy as jnp
from jax import lax
from jax.experimental import pallas as pl
from jax.experimental.pallas import tpu as pltpu
from jax.experimental.pallas import tpu_sc as plsc

NC = 2
NS = 16
NW = NC * NS
NB = 2


def _mm1_body(x_ref, w_ref, b_ref, o_ref):
    acc = jnp.dot(x_ref[...], w_ref[...], preferred_element_type=jnp.float32)
    o_ref[...] = jnp.maximum(acc + b_ref[...], 0.0)


def _mm2_body(p_ref, h_ref, w_ref, b_ref, o_ref):
    agg = p_ref[0] + p_ref[1]
    acc = jnp.dot(agg, w_ref[...], preferred_element_type=jnp.float32)
    o_ref[...] = jnp.maximum(acc + b_ref[...], 0.0) + h_ref[...]


def _sc_agg(h, src3, dst3, pad_n, k, ch0, ch1):
    n, d = h.shape
    chmax = max(ch0, ch1)
    rpt = pad_n // NS
    mesh = plsc.VectorSubcoreMesh(core_axis_name="c", subcore_axis_name="s")

    @functools.partial(
        pl.kernel,
        out_type=jax.ShapeDtypeStruct((NC, pad_n, d), jnp.float32),
        mesh=mesh,
        scratch_types=[
            pltpu.VMEM((chmax + NB, k), jnp.int32),
            pltpu.VMEM((k, d), jnp.float32),
            pltpu.VMEM((k, d), jnp.float32),
            pltpu.VMEM_SHARED((pad_n, d), jnp.float32),
            pltpu.SemaphoreType.DMA,
            pltpu.SemaphoreType.DMA,
            pltpu.SemaphoreType.DMA,
            pltpu.SemaphoreType.DMA,
        ],
    )
    def agg_kernel(h_hbm, src_hbm, dst_hbm, out_hbm,
                   src_v, rows0, rows1, shared,
                   gsem0, gsem1, dsem0, dsem1):
        cid = lax.axis_index("c")
        sid = lax.axis_index("s")
        row0 = sid * rpt
        chc = jnp.where(cid == 0, ch0, ch1)

        @pl.loop(0, k)
        def _(r):
            @pl.loop(0, d, step=16)
            def _(c0):
                rows0[r, pl.ds(c0, 16)] = jnp.zeros((16,), jnp.float32)

        nfull, rem = divmod(rpt, k)

        @pl.loop(0, nfull * k, step=k)
        def _(r):
            pltpu.sync_copy(rows0, shared.at[pl.ds(row0 + r, k)])

        if rem:
            pltpu.sync_copy(rows0.at[pl.ds(0, rem)],
                            shared.at[pl.ds(row0 + nfull * k, rem)])

        gwid = cid * NS + sid
        pltpu.sync_copy(src_hbm.at[gwid], src_v.at[pl.ds(0, chmax)])

        plsc.subcore_barrier()

        rows = (rows0, rows1)
        gsem = (gsem0, gsem1)
        dsem = (dsem0, dsem1)
        for b in range(NB):
            pltpu.async_copy(dst_hbm.at[gwid, b], src_v.at[chmax + b], dsem[b])
            pltpu.async_copy(h_hbm.at[src_v.at[b]], rows[b], gsem[b])

        @pl.loop(0, chc, step=NB)
        def _(j):
            for b in range(NB):
                pltpu.make_async_copy(dst_hbm.at[gwid, j + b],
                                      src_v.at[chmax + b], dsem[b]).wait()
                pltpu.make_async_copy(h_hbm.at[src_v.at[j + b]],
                                      rows[b], gsem[b]).wait()
                pltpu.sync_copy(rows[b], shared.at[src_v.at[chmax + b]],
                                add=True)

                @pl.when(j + b + NB < chc)
                def _():
                    pltpu.async_copy(dst_hbm.at[gwid, j + b + NB],
                                     src_v.at[chmax + b], dsem[b])
                    pltpu.async_copy(h_hbm.at[src_v.at[j + b + NB]],
                                     rows[b], gsem[b])

        plsc.subcore_barrier()

        pltpu.sync_copy(shared.at[pl.ds(row0, rpt)],
                        out_hbm.at[cid, pl.ds(row0, rpt)])

    return agg_kernel(h, src3, dst3)


def _split_edges(idx, e, k, ch0, ch1, pad_val):
    chmax = max(ch0, ch1)
    e0 = NS * ch0 * k
    cap1 = NS * ch1 * k
    p0 = idx[:e0].reshape(NS, ch0, k)
    p1 = jnp.concatenate(
        [idx[e0:], jnp.full((e0 + cap1 - e,), pad_val, jnp.int32)]
    ).reshape(NS, ch1, k)
    p0 = jnp.pad(p0, ((0, 0), (0, chmax - ch0), (0, 0)),
                 constant_values=pad_val)
    p1 = jnp.pad(p1, ((0, 0), (0, chmax - ch1), (0, 0)),
                 constant_values=pad_val)
    return jnp.concatenate([p0, p1], axis=0)


def kernel(x, edge_index, W1, b1, W2, b2):
    n, d = x.shape
    e = edge_index.shape[1]

    k = 88
    ch0, ch1 = 138, 90
    assert NS * (ch0 + ch1) * k >= e and ch0 % NB == 0 and ch1 % NB == 0
    pad_n = ((n + 127) // 128) * 128

    src = _split_edges(edge_index[0], e, k, ch0, ch1, 0)
    dst = _split_edges(edge_index[1], e, k, ch0, ch1, n)

    bn = 1000
    grid = (n // bn,)
    b1r = b1.reshape(1, d)
    b2r = b2.reshape(1, d)

    h = pl.pallas_call(
        _mm1_body,
        grid=grid,
        in_specs=[
            pl.BlockSpec((bn, d), lambda i: (i, 0)),
            pl.BlockSpec((d, d), lambda i: (0, 0)),
            pl.BlockSpec((1, d), lambda i: (0, 0)),
        ],
        out_specs=pl.BlockSpec((bn, d), lambda i: (i, 0)),
        out_shape=jax.ShapeDtypeStruct((n, d), jnp.float32),
    )(x, W1, b1r)

    partials = _sc_agg(h, src, dst, pad_n, k, ch0, ch1)

    out = pl.pallas_call(
        _mm2_body,
        grid=grid,
        in_specs=[
            pl.BlockSpec((NC, bn, d), lambda i: (0, i, 0)),
            pl.BlockSpec((bn, d), lambda i: (i, 0)),
            pl.BlockSpec((d, d), lambda i: (0, 0)),
            pl.BlockSpec((1, d), lambda i: (0, 0)),
        ],
        out_specs=pl.BlockSpec((bn, d), lambda i: (i, 0)),
        out_shape=jax.ShapeDtypeStruct((n, d), jnp.float32),
    )(partials, h, W2, b2r)

    return out


# speedup vs baseline: 1.1207x; 1.0124x over previous
import functools

import jax
import jax.numpy as jnp
from jax import lax
from jax.experimental import pallas as pl
from jax.experimental.pallas import tpu as pltpu
from jax.experimental.pallas import tpu_sc as plsc

NC = 2
NS = 16
NW = NC * NS
NB = 3


def _mm1_body(x_ref, w_ref, b_ref, o_ref):
    acc = jnp.dot(x_ref[...], w_ref[...], preferred_element_type=jnp.float32)
    o_ref[...] = jnp.maximum(acc + b_ref[...], 0.0)


def _mm2_body(p_ref, h_ref, w_ref, b_ref, o_ref):
    agg = p_ref[0] + p_ref[1]
    acc = jnp.dot(agg, w_ref[...], preferred_element_type=jnp.float32)
    o_ref[...] = jnp.maximum(acc + b_ref[...], 0.0) + h_ref[...]


def _sc_agg(h, src3, dst3, pad_n, k, ch0, ch1):
    n, d = h.shape
    chmax = max(ch0, ch1)
    rpt = pad_n // NS
    mesh = plsc.VectorSubcoreMesh(core_axis_name="c", subcore_axis_name="s")

    @functools.partial(
        pl.kernel,
        out_type=jax.ShapeDtypeStruct((NC, pad_n, d), jnp.float32),
        mesh=mesh,
        scratch_types=[
            pltpu.VMEM((chmax + NB, k), jnp.int32),
            pltpu.VMEM((k, d), jnp.float32),
            pltpu.VMEM((k, d), jnp.float32),
            pltpu.VMEM((k, d), jnp.float32),
            pltpu.VMEM_SHARED((pad_n, d), jnp.float32),
            pltpu.SemaphoreType.DMA,
            pltpu.SemaphoreType.DMA,
            pltpu.SemaphoreType.DMA,
            pltpu.SemaphoreType.DMA,
            pltpu.SemaphoreType.DMA,
            pltpu.SemaphoreType.DMA,
        ],
    )
    def agg_kernel(h_hbm, src_hbm, dst_hbm, out_hbm,
                   src_v, rows0, rows1, rows2, shared,
                   gsem0, gsem1, gsem2, dsem0, dsem1, dsem2):
        cid = lax.axis_index("c")
        sid = lax.axis_index("s")
        row0 = sid * rpt
        chc = jnp.where(cid == 0, ch0, ch1)

        @pl.loop(0, k)
        def _(r):
            @pl.loop(0, d, step=16)
            def _(c0):
                rows0[r, pl.ds(c0, 16)] = jnp.zeros((16,), jnp.float32)

        nfull, rem = divmod(rpt, k)

        @pl.loop(0, nfull * k, step=k)
        def _(r):
            pltpu.sync_copy(rows0, shared.at[pl.ds(row0 + r, k)])

        if rem:
            pltpu.sync_copy(rows0.at[pl.ds(0, rem)],
                            shared.at[pl.ds(row0 + nfull * k, rem)])

        gwid = cid * NS + sid
        pltpu.sync_copy(src_hbm.at[gwid], src_v.at[pl.ds(0, chmax)])

        plsc.subcore_barrier()

        rows = (rows0, rows1, rows2)
        gsem = (gsem0, gsem1, gsem2)
        dsem = (dsem0, dsem1, dsem2)
        for b in range(NB):
            pltpu.async_copy(dst_hbm.at[gwid, b], src_v.at[chmax + b], dsem[b])
            pltpu.async_copy(h_hbm.at[src_v.at[b]], rows[b], gsem[b])

        @pl.loop(0, chc, step=NB)
        def _(j):
            for b in range(NB):
                pltpu.make_async_copy(dst_hbm.at[gwid, j + b],
                                      src_v.at[chmax + b], dsem[b]).wait()
                pltpu.make_async_copy(h_hbm.at[src_v.at[j + b]],
                                      rows[b], gsem[b]).wait()
                pltpu.sync_copy(rows[b], shared.at[src_v.at[chmax + b]],
                                add=True)

                @pl.when(j + b + NB < chc)
                def _():
                    pltpu.async_copy(dst_hbm.at[gwid, j + b + NB],
                                     src_v.at[chmax + b], dsem[b])
                    pltpu.async_copy(h_hbm.at[src_v.at[j + b + NB]],
                                     rows[b], gsem[b])

        plsc.subcore_barrier()

        pltpu.sync_copy(shared.at[pl.ds(row0, rpt)],
                        out_hbm.at[cid, pl.ds(row0, rpt)])

    return agg_kernel(h, src3, dst3)


def _split_edges(idx, e, k, ch0, ch1, pad_val):
    chmax = max(ch0, ch1)
    e0 = NS * ch0 * k
    cap1 = NS * ch1 * k
    p0 = idx[:e0].reshape(NS, ch0, k)
    p1 = jnp.concatenate(
        [idx[e0:], jnp.full((e0 + cap1 - e,), pad_val, jnp.int32)]
    ).reshape(NS, ch1, k)
    p0 = jnp.pad(p0, ((0, 0), (0, chmax - ch0), (0, 0)),
                 constant_values=pad_val)
    p1 = jnp.pad(p1, ((0, 0), (0, chmax - ch1), (0, 0)),
                 constant_values=pad_val)
    return jnp.concatenate([p0, p1], axis=0)


def kernel(x, edge_index, W1, b1, W2, b2):
    n, d = x.shape
    e = edge_index.shape[1]

    k = 88
    ch0, ch1 = 117, 111
    assert NS * (ch0 + ch1) * k >= e and ch0 % NB == 0 and ch1 % NB == 0
    pad_n = ((n + 127) // 128) * 128

    src = _split_edges(edge_index[0], e, k, ch0, ch1, 0)
    dst = _split_edges(edge_index[1], e, k, ch0, ch1, n)

    bn = 1000
    grid = (n // bn,)
    b1r = b1.reshape(1, d)
    b2r = b2.reshape(1, d)

    h = pl.pallas_call(
        _mm1_body,
        grid=grid,
        in_specs=[
            pl.BlockSpec((bn, d), lambda i: (i, 0)),
            pl.BlockSpec((d, d), lambda i: (0, 0)),
            pl.BlockSpec((1, d), lambda i: (0, 0)),
        ],
        out_specs=pl.BlockSpec((bn, d), lambda i: (i, 0)),
        out_shape=jax.ShapeDtypeStruct((n, d), jnp.float32),
    )(x, W1, b1r)

    partials = _sc_agg(h, src, dst, pad_n, k, ch0, ch1)

    out = pl.pallas_call(
        _mm2_body,
        grid=grid,
        in_specs=[
            pl.BlockSpec((NC, bn, d), lambda i: (0, i, 0)),
            pl.BlockSpec((bn, d), lambda i: (i, 0)),
            pl.BlockSpec((d, d), lambda i: (0, 0)),
            pl.BlockSpec((1, d), lambda i: (0, 0)),
        ],
        out_specs=pl.BlockSpec((bn, d), lambda i: (i, 0)),
        out_shape=jax.ShapeDtypeStruct((n, d), jnp.float32),
    )(partials, h, W2, b2r)

    return out


# speedup vs baseline: 1.1633x; 1.0380x over previous
import functools

import jax
import jax.numpy as jnp
from jax import lax
from jax.experimental import pallas as pl
from jax.experimental.pallas import tpu as pltpu
from jax.experimental.pallas import tpu_sc as plsc

NC = 2
NS = 16
NW = NC * NS


def _mm1_body(x_ref, w_ref, b_ref, o_ref):
    acc = jnp.dot(x_ref[...], w_ref[...], preferred_element_type=jnp.float32)
    o_ref[...] = jnp.maximum(acc + b_ref[...], 0.0)


def _mm2_body(p_ref, h_ref, w_ref, b_ref, o_ref):
    agg = p_ref[0] + p_ref[1]
    acc = jnp.dot(agg, w_ref[...], preferred_element_type=jnp.float32)
    o_ref[...] = jnp.maximum(acc + b_ref[...], 0.0) + h_ref[...]


def _sc_agg(h, src3, dst3, pad_n, k, ch):
    n, d = h.shape
    rpt = pad_n // NS
    mesh = plsc.VectorSubcoreMesh(core_axis_name="c", subcore_axis_name="s")

    @functools.partial(
        pl.kernel,
        out_type=jax.ShapeDtypeStruct((NC, pad_n, d), jnp.float32),
        mesh=mesh,
        scratch_types=[
            pltpu.VMEM((ch + 3, k), jnp.int32),
            pltpu.VMEM((k, d), jnp.float32),
            pltpu.VMEM((k, d), jnp.float32),
            pltpu.VMEM((k, d), jnp.float32),
            pltpu.VMEM_SHARED((pad_n, d), jnp.float32),
            pltpu.SemaphoreType.DMA,
            pltpu.SemaphoreType.DMA,
            pltpu.SemaphoreType.DMA,
            pltpu.SemaphoreType.DMA,
            pltpu.SemaphoreType.DMA,
            pltpu.SemaphoreType.DMA,
        ],
    )
    def agg_kernel(h_hbm, src_hbm, dst_hbm, out_hbm,
                   src_v, rows0, rows1, rows2, shared,
                   gsem0, gsem1, gsem2, dsem0, dsem1, dsem2):
        cid = lax.axis_index("c")
        sid = lax.axis_index("s")
        row0 = sid * rpt

        @pl.loop(0, k)
        def _(r):
            @pl.loop(0, d, step=16)
            def _(c0):
                rows0[r, pl.ds(c0, 16)] = jnp.zeros((16,), jnp.float32)

        nfull, rem = divmod(rpt, k)

        @pl.loop(0, nfull * k, step=k)
        def _(r):
            pltpu.sync_copy(rows0, shared.at[pl.ds(row0 + r, k)])

        if rem:
            pltpu.sync_copy(rows0.at[pl.ds(0, rem)],
                            shared.at[pl.ds(row0 + nfull * k, rem)])

        gwid = sid * NC + cid
        pltpu.sync_copy(src_hbm.at[gwid], src_v.at[pl.ds(0, ch)])

        plsc.subcore_barrier()

        rows = (rows0, rows1, rows2)
        gsem = (gsem0, gsem1, gsem2)
        dsem = (dsem0, dsem1, dsem2)
        nb = len(rows)
        for b in range(nb):
            pltpu.async_copy(dst_hbm.at[gwid, b], src_v.at[ch + b], dsem[b])
            pltpu.async_copy(h_hbm.at[src_v.at[b]], rows[b], gsem[b])

        @pl.loop(0, ch, step=nb)
        def _(j):
            for b in range(nb):
                pltpu.make_async_copy(dst_hbm.at[gwid, j + b],
                                      src_v.at[ch + b], dsem[b]).wait()
                pltpu.make_async_copy(h_hbm.at[src_v.at[j + b]],
                                      rows[b], gsem[b]).wait()
                pltpu.sync_copy(rows[b], shared.at[src_v.at[ch + b]], add=True)

                @pl.when(j + b + nb < ch)
                def _():
                    pltpu.async_copy(dst_hbm.at[gwid, j + b + nb],
                                     src_v.at[ch + b], dsem[b])
                    pltpu.async_copy(h_hbm.at[src_v.at[j + b + nb]],
                                     rows[b], gsem[b])

        plsc.subcore_barrier()

        pltpu.sync_copy(shared.at[pl.ds(row0, rpt)],
                        out_hbm.at[cid, pl.ds(row0, rpt)])

    return agg_kernel(h, src3, dst3)


def kernel(x, edge_index, W1, b1, W2, b2):
    n, d = x.shape
    e = edge_index.shape[1]

    k = 88
    ch = -(-e // (NW * k * 3)) * 3
    e_pad = NW * ch * k
    pad_n = ((n + 127) // 128) * 128

    src = jnp.concatenate(
        [edge_index[0], jnp.zeros((e_pad - e,), jnp.int32)]).reshape(NW, ch, k)
    dst = jnp.concatenate(
        [edge_index[1], jnp.full((e_pad - e,), n, jnp.int32)]).reshape(NW, ch, k)

    bn = 1000
    grid = (n // bn,)
    b1r = b1.reshape(1, d)
    b2r = b2.reshape(1, d)

    h = pl.pallas_call(
        _mm1_body,
        grid=grid,
        in_specs=[
            pl.BlockSpec((bn, d), lambda i: (i, 0)),
            pl.BlockSpec((d, d), lambda i: (0, 0)),
            pl.BlockSpec((1, d), lambda i: (0, 0)),
        ],
        out_specs=pl.BlockSpec((bn, d), lambda i: (i, 0)),
        out_shape=jax.ShapeDtypeStruct((n, d), jnp.float32),
    )(x, W1, b1r)

    partials = _sc_agg(h, src, dst, pad_n, k, ch)

    out = pl.pallas_call(
        _mm2_body,
        grid=grid,
        in_specs=[
            pl.BlockSpec((NC, bn, d), lambda i: (0, i, 0)),
            pl.BlockSpec((bn, d), lambda i: (i, 0)),
            pl.BlockSpec((d, d), lambda i: (0, 0)),
            pl.BlockSpec((1, d), lambda i: (0, 0)),
        ],
        out_specs=pl.BlockSpec((bn, d), lambda i: (i, 0)),
        out_shape=jax.ShapeDtypeStruct((n, d), jnp.float32),
    )(partials, h, W2, b2r)

    return out
